# baseline (speedup 1.0000x reference)
import functools

import jax
import jax.numpy as jnp
import numpy as np
from jax.experimental import pallas as pl
from jax.experimental.pallas import tpu as pltpu
from jax.sharding import Mesh, PartitionSpec as P

_NE = 16
_H = 1024
_I = 2048
_T = 128


def _moe_body(start_ref, x_ref, rw_ref, ws_ref, w2s_ref, out_ref, dw_ref):
    e = pl.program_id(0)

    @pl.when(e == 0)
    def _router():
        logits = jax.lax.dot_general(
            x_ref[...], rw_ref[...], (((1,), (1,)), ((), ())),
            precision=jax.lax.Precision.HIGHEST,
            preferred_element_type=jnp.float32)
        m = jnp.max(logits, axis=1, keepdims=True)
        ex = jnp.exp(logits - m)
        probs = ex / jnp.sum(ex, axis=1, keepdims=True)
        pmax = jnp.max(probs, axis=1, keepdims=True)
        eids = jax.lax.broadcasted_iota(jnp.int32, (_T, _NE), 1)
        first = jnp.min(jnp.where(probs >= pmax, eids, _NE), axis=1,
                        keepdims=True)
        dw_ref[...] = jnp.where(eids == first, pmax, 0.0)
        out_ref[...] = jnp.zeros_like(out_ref)

    xb = x_ref[...].astype(jnp.bfloat16)
    wsb = ws_ref[0].astype(jnp.bfloat16)
    h = jax.lax.dot_general(
        xb, wsb, (((1,), (1,)), ((), ())),
        preferred_element_type=jnp.float32)
    gate = h[:, :_I]
    up = h[:, _I:]
    act = (gate * jax.lax.logistic(gate)) * up
    col = start_ref[0] + e
    eids = jax.lax.broadcasted_iota(jnp.int32, (_T, _NE), 1)
    we = jnp.sum(jnp.where(eids == col, dw_ref[...], 0.0), axis=1,
                 keepdims=True)
    actb = (act * we).astype(jnp.bfloat16)
    w2b = w2s_ref[0].astype(jnp.bfloat16)
    contrib = jax.lax.dot_general(
        actb, w2b, (((1,), (1,)), ((), ())),
        preferred_element_type=jnp.float32)
    out_ref[...] += contrib


def _run_experts(x, rw, ws_l, w2s_l, start):
    n_local = ws_l.shape[0]
    return pl.pallas_call(
        _moe_body,
        grid=(n_local,),
        in_specs=[
            pl.BlockSpec(memory_space=pltpu.SMEM),
            pl.BlockSpec((_T, _H), lambda e: (0, 0)),
            pl.BlockSpec((_NE, _H), lambda e: (0, 0)),
            pl.BlockSpec((1, 2 * _I, _H), lambda e: (e, 0, 0)),
            pl.BlockSpec((1, _H, _I), lambda e: (e, 0, 0)),
        ],
        out_specs=pl.BlockSpec((_T, _H), lambda e: (0, 0)),
        out_shape=jax.ShapeDtypeStruct((_T, _H), jnp.float32),
        scratch_shapes=[pltpu.VMEM((_T, _NE), jnp.float32)],
        compiler_params=pltpu.CompilerParams(
            dimension_semantics=("arbitrary",)),
    )(start, x, rw, ws_l, w2s_l)


def kernel(hidden_states, router_w, ws, w2s, top_k):
    scale = jnp.asarray(top_k, jnp.float32) / 1.0
    devs = jax.devices()
    if len(devs) >= 2:
        mesh = Mesh(np.array(devs[:2]), ("x",))

        @functools.partial(
            jax.shard_map, mesh=mesh, check_vma=False,
            in_specs=(P(), P(), P("x"), P("x")), out_specs=P())
        def _sharded(x, rw, ws_l, w2s_l):
            start = (jax.lax.axis_index("x") * (_NE // 2)).astype(jnp.int32)
            part = _run_experts(x, rw, ws_l, w2s_l,
                                jnp.reshape(start, (1,)))
            return jax.lax.psum(part, "x")

        out = _sharded(hidden_states, router_w, ws, w2s)
    else:
        out = _run_experts(hidden_states, router_w, ws, w2s,
                           jnp.zeros((1,), jnp.int32))
    return out * scale


# speedup vs baseline: 5.1993x; 5.1993x over previous
import jax
import jax.numpy as jnp
from jax.experimental import pallas as pl
from jax.experimental.pallas import tpu as pltpu

_NE = 16
_H = 1024
_I = 2048
_T = 128
_NJ = 2
_BI = _I // _NJ


def _moe_body(x_ref, rw_ref, wg_ref, wu_ref, w2s_ref, out_ref, dw_ref):
    e = pl.program_id(0)
    j = pl.program_id(1)

    @pl.when((e == 0) & (j == 0))
    def _router():
        logits = jax.lax.dot_general(
            x_ref[...], rw_ref[...], (((1,), (1,)), ((), ())),
            precision=jax.lax.Precision.HIGHEST,
            preferred_element_type=jnp.float32)
        m = jnp.max(logits, axis=1, keepdims=True)
        ex = jnp.exp(logits - m)
        probs = ex / jnp.sum(ex, axis=1, keepdims=True)
        pmax = jnp.max(probs, axis=1, keepdims=True)
        eids = jax.lax.broadcasted_iota(jnp.int32, (_T, _NE), 1)
        first = jnp.min(jnp.where(probs >= pmax, eids, _NE), axis=1,
                        keepdims=True)
        dw_ref[...] = jnp.where(eids == first, pmax, 0.0)
        out_ref[...] = jnp.zeros_like(out_ref)

    xb = x_ref[...].astype(jnp.bfloat16)
    gate = jax.lax.dot_general(
        xb, wg_ref[0].astype(jnp.bfloat16), (((1,), (1,)), ((), ())),
        preferred_element_type=jnp.float32)
    up = jax.lax.dot_general(
        xb, wu_ref[0].astype(jnp.bfloat16), (((1,), (1,)), ((), ())),
        preferred_element_type=jnp.float32)
    act = (gate * jax.lax.logistic(gate)) * up
    eids = jax.lax.broadcasted_iota(jnp.int32, (_T, _NE), 1)
    we = jnp.sum(jnp.where(eids == e, dw_ref[...], 0.0), axis=1,
                 keepdims=True)
    actb = (act * we).astype(jnp.bfloat16)
    contrib = jax.lax.dot_general(
        actb, w2s_ref[0].astype(jnp.bfloat16), (((1,), (1,)), ((), ())),
        preferred_element_type=jnp.float32)
    out_ref[...] += contrib


def kernel(hidden_states, router_w, ws, w2s, top_k):
    out = pl.pallas_call(
        _moe_body,
        grid=(_NE, _NJ),
        in_specs=[
            pl.BlockSpec((_T, _H), lambda e, j: (0, 0)),
            pl.BlockSpec((_NE, _H), lambda e, j: (0, 0)),
            pl.BlockSpec((1, _BI, _H), lambda e, j: (e, j, 0)),
            pl.BlockSpec((1, _BI, _H), lambda e, j: (e, j + _NJ, 0)),
            pl.BlockSpec((1, _H, _BI), lambda e, j: (e, 0, j)),
        ],
        out_specs=pl.BlockSpec((_T, _H), lambda e, j: (0, 0)),
        out_shape=jax.ShapeDtypeStruct((_T, _H), jnp.float32),
        scratch_shapes=[pltpu.VMEM((_T, _NE), jnp.float32)],
        compiler_params=pltpu.CompilerParams(
            dimension_semantics=("arbitrary", "arbitrary")),
    )(hidden_states, router_w, ws, ws, w2s)
    return out * (jnp.asarray(top_k, jnp.float32) / 1.0)
